# baseline (speedup 1.0000x reference)
import functools

import jax
import jax.numpy as jnp
from jax import lax
from jax.experimental import pallas as pl
from jax.experimental.pallas import tpu as pltpu
from jax.experimental.pallas import tpu_sc as plsc

_B = 4096
_T = 200
_D = 64
_NW = 32
_ROWS = _B // _NW
_TOK = _ROWS * _T
_LANES = 16
_NVR = _D // _LANES
_CH = 40
_NCH = _T // _CH


def _body(ids_hbm, msk_hbm, emb_hbm, out_hbm,
          idx_v, msk_v, rows_a, rows_b, rows_c, rows_d, outs_v, cnt_v,
          sem_a, sem_b, sem_c, sem_d):
    wid = lax.axis_index("s") * 2 + lax.axis_index("c")
    base = wid * _ROWS

    pltpu.sync_copy(ids_hbm.at[pl.ds(base, _ROWS)], idx_v)
    pltpu.sync_copy(msk_hbm.at[pl.ds(base, _ROWS)], msk_v)

    lanes = lax.iota(jnp.int32, 16)
    zerov = jnp.zeros((_LANES,), jnp.float32)

    def comp_row(r, c):
        off = lanes * 0
        rs = lanes * 0 + r
        for ci in range(_T // _LANES + 1):
            if ci == _T // _LANES:
                s = pl.ds(_T - _LANES, _LANES)
                tail_mask = lanes >= (_LANES - _T % _LANES)
            else:
                s = pl.ds(ci * _LANES, _LANES)
                tail_mask = None
            ids = idx_v[r, s]
            valid = msk_v[r, s] != 0
            if tail_mask is not None:
                valid = valid & tail_mask
            pos = off + plsc.cumsum(valid.astype(jnp.int32)) - 1
            plsc.store_scatter(idx_v, [rs, pos], ids, mask=valid)
            off = off + plsc.all_reduce_population_count(valid)
        cnt_v[r, pl.ds(0, _LANES)] = off
        return c

    lax.fori_loop(0, _ROWS, comp_row, 0)

    def row_copies(r, nch, rows_x, sem_x, do):
        def fj(j, c):
            cp = pltpu.make_async_copy(
                emb_hbm.at[idx_v.at[r, pl.ds(j * _CH, _CH)]],
                rows_x.at[pl.ds(j * _CH, _CH)], sem_x)
            if do == "start":
                cp.start()
            else:
                cp.wait()
            return c
        lax.fori_loop(0, nch, fj, 0)

    def cnt_of(r):
        return cnt_v[r, pl.ds(0, _LANES)][0]

    def fire(r, rows_x, sem_x):
        n = cnt_of(r)
        row_copies(r, (n + _CH - 1) // _CH, rows_x, sem_x, "start")

    def process(r, rows_x, sem_x):
        n = cnt_of(r)
        nch = (n + _CH - 1) // _CH
        row_copies(r, nch, rows_x, sem_x, "wait")

        n8 = (n + 7) // 8 * 8

        def zbody(j, c):
            for d in range(_NVR):
                rows_x[n + j, pl.ds(d * _LANES, _LANES)] = zerov
            return c
        lax.fori_loop(0, n8 - n, zbody, 0)

        def tbody(t8, accs):
            out = list(accs)
            for u in range(8):
                t = t8 * 8 + u
                for d in range(_NVR):
                    out[d] = out[d] + rows_x[t, pl.ds(d * _LANES, _LANES)]
            return tuple(out)

        accs = lax.fori_loop(0, n8 // 8, tbody, (zerov,) * _NVR)

        denom = n.astype(jnp.float32) + 1e-9
        for d in range(_NVR):
            outs_v[r, pl.ds(d * _LANES, _LANES)] = accs[d] / denom

    bufs = ((rows_a, sem_a), (rows_b, sem_b), (rows_c, sem_c),
            (rows_d, sem_d))
    for k in range(3):
        fire(k, *bufs[k])

    def row_body(i, c):
        r0 = i * 4
        for k in range(4):
            r = r0 + k

            @pl.when(r + 3 < _ROWS)
            def _():
                fire(r + 3, *bufs[(k + 3) % 4])

            process(r, *bufs[k])
        return c

    lax.fori_loop(0, _ROWS // 4, row_body, 0)

    pltpu.sync_copy(outs_v, out_hbm.at[pl.ds(base, _ROWS)])


@functools.partial(jax.jit, donate_argnums=())
def _pooled(ids_flat, msk_flat, emb):
    mesh = plsc.VectorSubcoreMesh(core_axis_name="c", subcore_axis_name="s")
    call = pl.kernel(
        _body,
        out_type=jax.ShapeDtypeStruct((_B, _D), jnp.float32),
        mesh=mesh,
        compiler_params=pltpu.CompilerParams(
            needs_layout_passes=False, use_tc_tiling_on_sc=False),
        scratch_types=[
            pltpu.VMEM((_ROWS, _T), jnp.int32),
            pltpu.VMEM((_ROWS, _T), jnp.int32),
            pltpu.VMEM((_T, _D), jnp.float32),
            pltpu.VMEM((_T, _D), jnp.float32),
            pltpu.VMEM((_T, _D), jnp.float32),
            pltpu.VMEM((_T, _D), jnp.float32),
            pltpu.VMEM((_ROWS, _D), jnp.float32),
            pltpu.VMEM((_ROWS, _LANES), jnp.int32),
            pltpu.SemaphoreType.DMA,
            pltpu.SemaphoreType.DMA,
            pltpu.SemaphoreType.DMA,
            pltpu.SemaphoreType.DMA,
        ],
    )
    return call(ids_flat, msk_flat, emb)


def kernel(input_ids, attention_mask, emb):
    return _pooled(input_ids.astype(jnp.int32),
                   attention_mask.astype(jnp.int32), emb)


# speedup vs baseline: 1.0177x; 1.0177x over previous
import functools

import jax
import jax.numpy as jnp
from jax import lax
from jax.experimental import pallas as pl
from jax.experimental.pallas import tpu as pltpu
from jax.experimental.pallas import tpu_sc as plsc

_B = 4096
_T = 200
_D = 64
_NW = 32
_ROWS = _B // _NW
_TOK = _ROWS * _T
_LANES = 16
_NVR = _D // _LANES
_CH = 40
_NCH = _T // _CH


def _body(ids_hbm, msk_hbm, emb_hbm, out_hbm,
          idx_v, msk_v, rows_a, rows_b, rows_c, rows_d, outs_v, cnt_v,
          sem_a, sem_b, sem_c, sem_d):
    wid = lax.axis_index("s") * 2 + lax.axis_index("c")
    base = wid * _ROWS
    tb = base * _T

    pltpu.sync_copy(ids_hbm.at[pl.ds(tb, _TOK)], idx_v)
    pltpu.sync_copy(msk_hbm.at[pl.ds(tb, _TOK)], msk_v)

    lanes = lax.iota(jnp.int32, 16)
    zerov = jnp.zeros((_LANES,), jnp.float32)

    def comp_row(r, c):
        row0 = r * _T
        off = lanes * 0 + row0
        for ci in range(_T // _LANES + 1):
            s = pl.ds(row0 + ci * _LANES, _LANES)
            ids = idx_v[s]
            valid = msk_v[s] != 0
            if ci == _T // _LANES:
                valid = valid & (lanes < _T % _LANES)
            pos = off + plsc.cumsum(valid.astype(jnp.int32)) - 1
            plsc.store_scatter(idx_v, [pos], ids, mask=valid)
            off = off + plsc.all_reduce_population_count(valid)
        cnt_v[r, pl.ds(0, _LANES)] = off - row0
        return c

    lax.fori_loop(0, _ROWS, comp_row, 0)

    def row_copies(r, nch, rows_x, sem_x, do):
        def fj(j, c):
            cp = pltpu.make_async_copy(
                emb_hbm.at[idx_v.at[pl.ds(r * _T + j * _CH, _CH)]],
                rows_x.at[pl.ds(j * _CH, _CH)], sem_x)
            if do == "start":
                cp.start()
            else:
                cp.wait()
            return c
        lax.fori_loop(0, nch, fj, 0)

    def cnt_of(r):
        return cnt_v[r, pl.ds(0, _LANES)][0]

    def fire(r, rows_x, sem_x):
        n = cnt_of(r)
        row_copies(r, (n + _CH - 1) // _CH, rows_x, sem_x, "start")

    def process(r, rows_x, sem_x):
        n = cnt_of(r)
        nch = (n + _CH - 1) // _CH
        row_copies(r, nch, rows_x, sem_x, "wait")

        n8 = (n + 7) // 8 * 8

        def zbody(j, c):
            for d in range(_NVR):
                rows_x[n + j, pl.ds(d * _LANES, _LANES)] = zerov
            return c
        lax.fori_loop(0, n8 - n, zbody, 0)

        def tbody(t8, accs):
            out = list(accs)
            for u in range(8):
                t = t8 * 8 + u
                for d in range(_NVR):
                    out[d] = out[d] + rows_x[t, pl.ds(d * _LANES, _LANES)]
            return tuple(out)

        accs = lax.fori_loop(0, n8 // 8, tbody, (zerov,) * _NVR)

        denom = zerov + (n.astype(jnp.float32) + 1e-9)
        inv = (zerov + 1.0) / denom
        for d in range(_NVR):
            outs_v[r, pl.ds(d * _LANES, _LANES)] = accs[d] * inv

    bufs = ((rows_a, sem_a), (rows_b, sem_b), (rows_c, sem_c),
            (rows_d, sem_d))
    for k in range(3):
        fire(k, *bufs[k])

    def row_body(i, c):
        r0 = i * 4
        for k in range(4):
            r = r0 + k

            @pl.when(r + 3 < _ROWS)
            def _():
                fire(r + 3, *bufs[(k + 3) % 4])

            process(r, *bufs[k])
        return c

    lax.fori_loop(0, _ROWS // 4, row_body, 0)

    pltpu.sync_copy(outs_v, out_hbm.at[pl.ds(base, _ROWS)])


@functools.partial(jax.jit, donate_argnums=())
def _pooled(ids_flat, msk_flat, emb):
    mesh = plsc.VectorSubcoreMesh(core_axis_name="c", subcore_axis_name="s")
    call = pl.kernel(
        _body,
        out_type=jax.ShapeDtypeStruct((_B, _D), jnp.float32),
        mesh=mesh,
        compiler_params=pltpu.CompilerParams(
            needs_layout_passes=False, use_tc_tiling_on_sc=False),
        scratch_types=[
            pltpu.VMEM((_TOK,), jnp.int32),
            pltpu.VMEM((_TOK,), jnp.int32),
            pltpu.VMEM((_T, _D), jnp.float32),
            pltpu.VMEM((_T, _D), jnp.float32),
            pltpu.VMEM((_T, _D), jnp.float32),
            pltpu.VMEM((_T, _D), jnp.float32),
            pltpu.VMEM((_ROWS, _D), jnp.float32),
            pltpu.VMEM((_ROWS, _LANES), jnp.int32),
            pltpu.SemaphoreType.DMA,
            pltpu.SemaphoreType.DMA,
            pltpu.SemaphoreType.DMA,
            pltpu.SemaphoreType.DMA,
        ],
    )
    return call(ids_flat, msk_flat, emb)


def kernel(input_ids, attention_mask, emb):
    ids = input_ids.reshape(-1).astype(jnp.int32)
    msk = attention_mask.reshape(-1).astype(jnp.int32)
    return _pooled(ids, msk, emb)


# speedup vs baseline: 1.0278x; 1.0099x over previous
import functools

import jax
import jax.numpy as jnp
from jax import lax
from jax.experimental import pallas as pl
from jax.experimental.pallas import tpu as pltpu
from jax.experimental.pallas import tpu_sc as plsc

_B = 4096
_T = 200
_D = 64
_NW = 32
_ROWS = _B // _NW
_TOK = _ROWS * _T
_LANES = 16
_NVR = _D // _LANES
_CH = 40
_NCH = _T // _CH


def _body(ids_hbm, msk_hbm, emb_hbm, out_hbm,
          idx_v, msk_v, rows_a, rows_b, rows_c, rows_d, outs_v, cnt_v,
          sem_a, sem_b, sem_c, sem_d):
    wid = lax.axis_index("s") * 2 + lax.axis_index("c")
    base = wid * _ROWS
    tb = base * _T

    pltpu.sync_copy(ids_hbm.at[pl.ds(tb, _TOK)], idx_v)
    pltpu.sync_copy(msk_hbm.at[pl.ds(tb, _TOK)], msk_v)

    lanes = lax.iota(jnp.int32, 16)
    zerov = jnp.zeros((_LANES,), jnp.float32)

    def comp_row(r, c):
        row0 = r * _T
        off = lanes * 0 + row0
        for ci in range(_T // _LANES + 1):
            s = pl.ds(row0 + ci * _LANES, _LANES)
            ids = idx_v[s]
            valid = msk_v[s] != 0
            if ci == _T // _LANES:
                valid = valid & (lanes < _T % _LANES)
            pos = off + plsc.cumsum(valid.astype(jnp.int32)) - 1
            plsc.store_scatter(idx_v, [pos], ids, mask=valid)
            off = off + plsc.all_reduce_population_count(valid)
        cnt_v[r, pl.ds(0, _LANES)] = off - row0
        return c

    def row_copies(r, nch, rows_x, sem_x, do):
        def fj(j, c):
            cp = pltpu.make_async_copy(
                emb_hbm.at[idx_v.at[pl.ds(r * _T + j * _CH, _CH)]],
                rows_x.at[pl.ds(j * _CH, _CH)], sem_x)
            if do == "start":
                cp.start()
            else:
                cp.wait()
            return c
        lax.fori_loop(0, nch, fj, 0)

    def cnt_of(r):
        return cnt_v[r, pl.ds(0, _LANES)][0]

    def fire(r, rows_x, sem_x):
        n = cnt_of(r)
        row_copies(r, (n + _CH - 1) // _CH, rows_x, sem_x, "start")

    def process(r, rows_x, sem_x):
        n = cnt_of(r)
        nch = (n + _CH - 1) // _CH
        row_copies(r, nch, rows_x, sem_x, "wait")

        n8 = (n + 7) // 8 * 8

        def zbody(j, c):
            for d in range(_NVR):
                rows_x[n + j, pl.ds(d * _LANES, _LANES)] = zerov
            return c
        lax.fori_loop(0, n8 - n, zbody, 0)

        def tbody(t8, accs):
            out = list(accs)
            for u in range(8):
                t = t8 * 8 + u
                for d in range(_NVR):
                    out[d] = out[d] + rows_x[t, pl.ds(d * _LANES, _LANES)]
            return tuple(out)

        accs = lax.fori_loop(0, n8 // 8, tbody, (zerov,) * _NVR)

        denom = zerov + (n.astype(jnp.float32) + 1e-9)
        inv = (zerov + 1.0) / denom
        for d in range(_NVR):
            outs_v[r, pl.ds(d * _LANES, _LANES)] = accs[d] * inv

    bufs = ((rows_a, sem_a), (rows_b, sem_b), (rows_c, sem_c),
            (rows_d, sem_d))
    lax.fori_loop(0, 4, comp_row, 0)
    for k in range(3):
        fire(k, *bufs[k])

    def row_body(i, c):
        r0 = i * 4
        for k in range(4):
            r = r0 + k

            @pl.when(r + 4 < _ROWS)
            def _():
                comp_row(r + 4, 0)

            @pl.when(r + 3 < _ROWS)
            def _():
                fire(r + 3, *bufs[(k + 3) % 4])

            process(r, *bufs[k])
        return c

    lax.fori_loop(0, _ROWS // 4, row_body, 0)

    pltpu.sync_copy(outs_v, out_hbm.at[pl.ds(base, _ROWS)])


@functools.partial(jax.jit, donate_argnums=())
def _pooled(ids_flat, msk_flat, emb):
    mesh = plsc.VectorSubcoreMesh(core_axis_name="c", subcore_axis_name="s")
    call = pl.kernel(
        _body,
        out_type=jax.ShapeDtypeStruct((_B, _D), jnp.float32),
        mesh=mesh,
        compiler_params=pltpu.CompilerParams(
            needs_layout_passes=False, use_tc_tiling_on_sc=False),
        scratch_types=[
            pltpu.VMEM((_TOK,), jnp.int32),
            pltpu.VMEM((_TOK,), jnp.int32),
            pltpu.VMEM((_T, _D), jnp.float32),
            pltpu.VMEM((_T, _D), jnp.float32),
            pltpu.VMEM((_T, _D), jnp.float32),
            pltpu.VMEM((_T, _D), jnp.float32),
            pltpu.VMEM((_ROWS, _D), jnp.float32),
            pltpu.VMEM((_ROWS, _LANES), jnp.int32),
            pltpu.SemaphoreType.DMA,
            pltpu.SemaphoreType.DMA,
            pltpu.SemaphoreType.DMA,
            pltpu.SemaphoreType.DMA,
        ],
    )
    return call(ids_flat, msk_flat, emb)


def kernel(input_ids, attention_mask, emb):
    ids = input_ids.reshape(-1).astype(jnp.int32)
    msk = attention_mask.reshape(-1).astype(jnp.int32)
    return _pooled(ids, msk, emb)
